# baseline (speedup 1.0000x reference)
import functools

import jax
import jax.numpy as jnp
from jax import lax
from jax.experimental import pallas as pl
from jax.experimental.pallas import tpu as pltpu
from jax.experimental.pallas import tpu_sc as plsc

BATCH = 1024
SEQ = 77
HIDDEN = 768
ROWS = BATCH * SEQ
NUM_WORKERS = 32
ROWS_PER_WORKER = ROWS // NUM_WORKERS
CHUNK = 32
NUM_CHUNKS = ROWS_PER_WORKER // CHUNK
LANES = 16
VECS_PER_ROW = HIDDEN // LANES


def _emb_body(ids_hbm, tok_hbm, pos_hbm, out_hbm, idx_v, pos_v, buf, sem):
    wid = lax.axis_index("s") * 2 + lax.axis_index("c")
    base = wid * ROWS_PER_WORKER
    pltpu.sync_copy(ids_hbm.at[pl.ds(base, ROWS_PER_WORKER)], idx_v)
    pltpu.sync_copy(pos_hbm, pos_v)

    def chunk_body(j, carry):
        row0 = base + j * CHUNK
        r0 = lax.rem(j * CHUNK, SEQ)
        pltpu.async_copy(
            tok_hbm.at[idx_v.at[pl.ds(j * CHUNK, CHUNK)]], buf, sem
        ).wait()

        def row_body(i, c2):
            rr = r0 + i
            rr = jnp.where(rr >= SEQ, rr - SEQ, rr)
            for g in range(VECS_PER_ROW):
                v = pos_v[rr, pl.ds(g * LANES, LANES)]
                plsc.addupdate(buf.at[i, pl.ds(g * LANES, LANES)], v)
            return c2

        lax.fori_loop(0, CHUNK, row_body, 0)
        pltpu.sync_copy(buf, out_hbm.at[pl.ds(row0, CHUNK)])
        return carry

    lax.fori_loop(0, NUM_CHUNKS, chunk_body, 0)


@functools.partial(
    pl.kernel,
    out_type=jax.ShapeDtypeStruct((ROWS, HIDDEN), jnp.float32),
    mesh=plsc.VectorSubcoreMesh(core_axis_name="c", subcore_axis_name="s"),
    scratch_types=[
        pltpu.VMEM((ROWS_PER_WORKER,), jnp.int32),
        pltpu.VMEM((SEQ, HIDDEN), jnp.float32),
        pltpu.VMEM((CHUNK, HIDDEN), jnp.float32),
        pltpu.SemaphoreType.DMA,
    ],
)
def _emb_kernel(ids_hbm, tok_hbm, pos_hbm, out_hbm, idx_v, pos_v, buf, sem):
    _emb_body(ids_hbm, tok_hbm, pos_hbm, out_hbm, idx_v, pos_v, buf, sem)


def kernel(input_ids, token_table, position_table):
    ids_flat = input_ids.reshape(-1).astype(jnp.int32)
    out = _emb_kernel(ids_flat, token_table, position_table)
    return out.reshape(BATCH, SEQ, HIDDEN)


# speedup vs baseline: 2.7155x; 2.7155x over previous
import functools

import jax
import jax.numpy as jnp
from jax import lax
from jax.experimental import pallas as pl
from jax.experimental.pallas import tpu as pltpu
from jax.experimental.pallas import tpu_sc as plsc

BATCH = 1024
SEQ = 77
HIDDEN = 768
ROWS = BATCH * SEQ
NUM_WORKERS = 32
WB = BATCH // NUM_WORKERS
IDS_PER_WORKER = WB * SEQ
LANES = 16
VECS_PER_ROW = HIDDEN // LANES


def _gather(tok_hbm, idx_v, s, buf, sem):
    return pltpu.async_copy(tok_hbm.at[idx_v.at[pl.ds(s * WB, WB)]], buf, sem)


def _scatter(out_hbm, row0, buf, sem):
    return pltpu.async_copy(buf, out_hbm.at[pl.ds(row0, WB)], sem)


def _wait_scatter(out_hbm, buf, sem):
    pltpu.make_async_copy(buf, out_hbm.at[pl.ds(0, WB)], sem).wait()


def _add_pos(pos_v, s, buf):
    for g in range(VECS_PER_ROW):
        v = pos_v[pl.ds(s * HIDDEN + g * LANES, LANES)]
        for i in range(WB):
            plsc.addupdate(buf.at[i, pl.ds(g * LANES, LANES)], v)


def _emb_body(ids_hbm, tok_hbm, pos_hbm, out_hbm,
              idx_v, pos_v, buf0, buf1, gsem0, gsem1, ssem0, ssem1):
    wid = lax.axis_index("s") * 2 + lax.axis_index("c")
    pltpu.sync_copy(ids_hbm.at[pl.ds(wid * IDS_PER_WORKER, IDS_PER_WORKER)],
                    idx_v)
    pltpu.sync_copy(pos_hbm, pos_v)
    wb0 = wid * WB

    def pair_body(k, carry):
        a = 2 * k
        b = 2 * k + 1

        @pl.when(k > 0)
        def _():
            _wait_scatter(out_hbm, buf0, ssem0)
        g0 = _gather(tok_hbm, idx_v, a, buf0, gsem0)

        @pl.when(k > 0)
        def _():
            _wait_scatter(out_hbm, buf1, ssem1)
        g1 = _gather(tok_hbm, idx_v, b, buf1, gsem1)

        g0.wait()
        _add_pos(pos_v, a, buf0)
        _scatter(out_hbm, a * BATCH + wb0, buf0, ssem0)

        g1.wait()
        _add_pos(pos_v, b, buf1)
        _scatter(out_hbm, b * BATCH + wb0, buf1, ssem1)
        return carry

    lax.fori_loop(0, (SEQ - 1) // 2, pair_body, 0)

    s_t = SEQ - 1
    _wait_scatter(out_hbm, buf0, ssem0)
    _gather(tok_hbm, idx_v, s_t, buf0, gsem0).wait()
    _add_pos(pos_v, s_t, buf0)
    _scatter(out_hbm, s_t * BATCH + wb0, buf0, ssem0)
    _wait_scatter(out_hbm, buf0, ssem0)
    _wait_scatter(out_hbm, buf1, ssem1)


@functools.partial(
    pl.kernel,
    out_type=jax.ShapeDtypeStruct((ROWS, HIDDEN), jnp.float32),
    mesh=plsc.VectorSubcoreMesh(core_axis_name="c", subcore_axis_name="s"),
    scratch_types=[
        pltpu.VMEM((IDS_PER_WORKER,), jnp.int32),
        pltpu.VMEM((SEQ * HIDDEN,), jnp.float32),
        pltpu.VMEM((WB, HIDDEN), jnp.float32),
        pltpu.VMEM((WB, HIDDEN), jnp.float32),
        pltpu.SemaphoreType.DMA,
        pltpu.SemaphoreType.DMA,
        pltpu.SemaphoreType.DMA,
        pltpu.SemaphoreType.DMA,
    ],
)
def _emb_kernel(ids_hbm, tok_hbm, pos_hbm, out_hbm,
                idx_v, pos_v, buf0, buf1, gsem0, gsem1, ssem0, ssem1):
    _emb_body(ids_hbm, tok_hbm, pos_hbm, out_hbm,
              idx_v, pos_v, buf0, buf1, gsem0, gsem1, ssem0, ssem1)


def kernel(input_ids, token_table, position_table):
    ids_grouped = (
        input_ids.astype(jnp.int32)
        .reshape(NUM_WORKERS, WB, SEQ)
        .transpose(0, 2, 1)
        .reshape(-1)
    )
    out = _emb_kernel(ids_grouped, token_table, position_table.reshape(-1))
    return out.reshape(SEQ, BATCH, HIDDEN).transpose(1, 0, 2)


# speedup vs baseline: 5.1788x; 1.9071x over previous
import functools

import jax
import jax.numpy as jnp
from jax import lax
from jax.experimental import pallas as pl
from jax.experimental.pallas import tpu as pltpu
from jax.experimental.pallas import tpu_sc as plsc

BATCH = 1024
SEQ = 77
HIDDEN = 768
ROWS = BATCH * SEQ
NUM_WORKERS = 32
WB = BATCH // NUM_WORKERS
IDS_PER_WORKER = WB * SEQ
LANES = 16
NBUF = 4


def _gather(tok_hbm, idx_v, u, buf, sem):
    return pltpu.async_copy(tok_hbm.at[idx_v.at[pl.ds(u * WB, WB)]], buf, sem)


def _add_pos(posbuf, buf):
    def g_body(g, carry):
        o = g * 2 * LANES
        v0 = posbuf[pl.ds(o, LANES)]
        v1 = posbuf[pl.ds(o + LANES, LANES)]
        for i in range(WB):
            plsc.addupdate(buf.at[i, pl.ds(o, LANES)], v0)
            plsc.addupdate(buf.at[i, pl.ds(o + LANES, LANES)], v1)
        return carry

    lax.fori_loop(0, HIDDEN // (2 * LANES), g_body, 0)


def _emb_body(ids_hbm, tok_hbm, pos_hbm, out_hbm,
              idx_v, posbuf, bufs, pos_sh, gsems, ssems):
    cid = lax.axis_index("c")
    wid = lax.axis_index("s") * 2 + cid
    wb0 = wid * WB
    pltpu.sync_copy(ids_hbm.at[pl.ds(wid * IDS_PER_WORKER, IDS_PER_WORKER)],
                    idx_v)
    for u in range(NBUF - 1):
        _gather(tok_hbm, idx_v, u, bufs[u], gsems[u])
    @pl.when(lax.axis_index("s") == 0)
    def _():
        pltpu.sync_copy(pos_hbm, pos_sh)
    plsc.subcore_barrier()

    def sub_body(c, q, i):
        bn = (q + NBUF - 1) % NBUF
        def wait_prev_scatter():
            pltpu.make_async_copy(
                bufs[bn], out_hbm.at[pl.ds(wb0, WB)], ssems[bn]).wait()

        if q == 0:
            @pl.when(i > 0)
            def _():
                wait_prev_scatter()
        else:
            wait_prev_scatter()

        u_pre = c + NBUF - 1

        def prefetch():
            _gather(tok_hbm, idx_v, u_pre, bufs[bn], gsems[bn])

        if q < 2:
            prefetch()
        else:
            @pl.when(u_pre < SEQ)
            def _():
                prefetch()

        pltpu.sync_copy(pos_sh.at[pl.ds(c * HIDDEN, HIDDEN)], posbuf)
        pltpu.make_async_copy(
            tok_hbm.at[idx_v.at[pl.ds(c * WB, WB)]], bufs[q], gsems[q]).wait()
        _add_pos(posbuf, bufs[q])
        pltpu.async_copy(
            bufs[q], out_hbm.at[pl.ds(c * BATCH + wb0, WB)], ssems[q])

    def quad_body(i, carry):
        for q in range(NBUF):
            sub_body(i * NBUF + q, q, i)
        return carry

    lax.fori_loop(0, (SEQ - 1) // NBUF, quad_body, 0)

    c_t = SEQ - 1
    pltpu.make_async_copy(
        bufs[NBUF - 1], out_hbm.at[pl.ds(wb0, WB)], ssems[NBUF - 1]).wait()
    pltpu.sync_copy(pos_sh.at[pl.ds(c_t * HIDDEN, HIDDEN)], posbuf)
    pltpu.make_async_copy(
        tok_hbm.at[idx_v.at[pl.ds(c_t * WB, WB)]], bufs[0], gsems[0]).wait()
    _add_pos(posbuf, bufs[0])
    pltpu.async_copy(
        bufs[0], out_hbm.at[pl.ds(c_t * BATCH + wb0, WB)], ssems[0])
    pltpu.make_async_copy(
        bufs[0], out_hbm.at[pl.ds(wb0, WB)], ssems[0]).wait()


@functools.partial(
    pl.kernel,
    out_type=jax.ShapeDtypeStruct((ROWS, HIDDEN), jnp.float32),
    mesh=plsc.VectorSubcoreMesh(core_axis_name="c", subcore_axis_name="s"),
    scratch_types=[
        pltpu.VMEM((IDS_PER_WORKER,), jnp.int32),
        pltpu.VMEM((HIDDEN,), jnp.float32),
        pltpu.VMEM((WB, HIDDEN), jnp.float32),
        pltpu.VMEM((WB, HIDDEN), jnp.float32),
        pltpu.VMEM((WB, HIDDEN), jnp.float32),
        pltpu.VMEM((WB, HIDDEN), jnp.float32),
        pltpu.VMEM_SHARED((SEQ * HIDDEN,), jnp.float32),
        pltpu.SemaphoreType.DMA,
        pltpu.SemaphoreType.DMA,
        pltpu.SemaphoreType.DMA,
        pltpu.SemaphoreType.DMA,
        pltpu.SemaphoreType.DMA,
        pltpu.SemaphoreType.DMA,
        pltpu.SemaphoreType.DMA,
        pltpu.SemaphoreType.DMA,
    ],
)
def _emb_kernel(ids_hbm, tok_hbm, pos_hbm, out_hbm,
                idx_v, posbuf, b0, b1, b2, b3, pos_sh,
                g0, g1, g2, g3, s0, s1, s2, s3):
    _emb_body(ids_hbm, tok_hbm, pos_hbm, out_hbm,
              idx_v, posbuf, (b0, b1, b2, b3), pos_sh,
              (g0, g1, g2, g3), (s0, s1, s2, s3))


def kernel(input_ids, token_table, position_table):
    ids_grouped = (
        input_ids.astype(jnp.int32)
        .reshape(NUM_WORKERS, WB, SEQ)
        .transpose(0, 2, 1)
        .reshape(-1)
    )
    out = _emb_kernel(ids_grouped, token_table, position_table.reshape(-1))
    return out.reshape(SEQ, BATCH, HIDDEN).transpose(1, 0, 2)


# speedup vs baseline: 5.2520x; 1.0141x over previous
import functools

import jax
import jax.numpy as jnp
from jax import lax
from jax.experimental import pallas as pl
from jax.experimental.pallas import tpu as pltpu
from jax.experimental.pallas import tpu_sc as plsc

BATCH = 1024
SEQ = 77
HIDDEN = 768
ROWS = BATCH * SEQ
NUM_WORKERS = 32
WB = BATCH // NUM_WORKERS
IDS_PER_WORKER = WB * SEQ
LANES = 16
NPAIR = SEQ // 2


def _gather_pair(tok_hbm, idx_v, p, buf, h, sem):
    return pltpu.async_copy(
        tok_hbm.at[idx_v.at[pl.ds(p * 2 * WB, 2 * WB)]],
        buf.at[pl.ds(h * 2 * WB, 2 * WB)], sem)


def _wait_scatter(out_hbm, buf, wb0, sem):
    pltpu.make_async_copy(
        buf.at[pl.ds(0, WB)], out_hbm.at[pl.ds(wb0, WB)], sem).wait()


def _add_pos(posbuf, pq, buf, row0):
    def g_body(g, carry):
        o = g * 2 * LANES
        v0 = posbuf[pl.ds(pq * HIDDEN + o, LANES)]
        v1 = posbuf[pl.ds(pq * HIDDEN + o + LANES, LANES)]
        for i in range(WB):
            plsc.addupdate(buf.at[row0 + i, pl.ds(o, LANES)], v0)
            plsc.addupdate(buf.at[row0 + i, pl.ds(o + LANES, LANES)], v1)
        return carry

    lax.fori_loop(0, HIDDEN // (2 * LANES), g_body, 0)


def _emb_body(ids_hbm, tok_hbm, pos_hbm, out_hbm,
              idx_v, posbuf, buf, pos_sh, gsems, ssems):
    wid = lax.axis_index("s") * 2 + lax.axis_index("c")
    wb0 = wid * WB
    pltpu.sync_copy(ids_hbm.at[pl.ds(wid * IDS_PER_WORKER, IDS_PER_WORKER)],
                    idx_v)
    _gather_pair(tok_hbm, idx_v, 0, buf, 0, gsems[0])
    @pl.when(lax.axis_index("s") == 0)
    def _():
        pltpu.sync_copy(pos_hbm, pos_sh)
    plsc.subcore_barrier()

    def sub_body(p, h, i):
        hn = 1 - h

        def wait_and_prefetch():
            _wait_scatter(out_hbm, buf, wb0, ssems[hn])
            _wait_scatter(out_hbm, buf, wb0, ssems[hn])
            _gather_pair(tok_hbm, idx_v, p + 1, buf, hn, gsems[hn])

        if h == 0:
            @pl.when(i > 0)
            def _():
                wait_and_prefetch()

            @pl.when(i == 0)
            def _():
                _gather_pair(tok_hbm, idx_v, p + 1, buf, hn, gsems[hn])
        else:
            @pl.when(i < (NPAIR // 2) - 1)
            def _():
                wait_and_prefetch()

        pltpu.sync_copy(pos_sh.at[pl.ds(p * 2 * HIDDEN, 2 * HIDDEN)], posbuf)
        pltpu.make_async_copy(
            tok_hbm.at[idx_v.at[pl.ds(p * 2 * WB, 2 * WB)]],
            buf.at[pl.ds(h * 2 * WB, 2 * WB)], gsems[h]).wait()
        for pq in range(2):
            row0 = h * 2 * WB + pq * WB
            _add_pos(posbuf, pq, buf, row0)
            pltpu.async_copy(
                buf.at[pl.ds(row0, WB)],
                out_hbm.at[pl.ds((2 * p + pq) * BATCH + wb0, WB)], ssems[h])

    def duo_body(i, carry):
        sub_body(2 * i, 0, i)
        sub_body(2 * i + 1, 1, i)
        return carry

    lax.fori_loop(0, NPAIR // 2, duo_body, 0)

    c_t = SEQ - 1
    _wait_scatter(out_hbm, buf, wb0, ssems[0])
    _wait_scatter(out_hbm, buf, wb0, ssems[0])
    pltpu.async_copy(
        tok_hbm.at[idx_v.at[pl.ds(c_t * WB, WB)]],
        buf.at[pl.ds(0, WB)], gsems[0])
    pltpu.sync_copy(pos_sh.at[pl.ds(c_t * HIDDEN, HIDDEN)],
                    posbuf.at[pl.ds(0, HIDDEN)])
    pltpu.make_async_copy(
        tok_hbm.at[idx_v.at[pl.ds(c_t * WB, WB)]],
        buf.at[pl.ds(0, WB)], gsems[0]).wait()
    _add_pos(posbuf, 0, buf, 0)
    pltpu.async_copy(
        buf.at[pl.ds(0, WB)],
        out_hbm.at[pl.ds(c_t * BATCH + wb0, WB)], ssems[0])
    _wait_scatter(out_hbm, buf, wb0, ssems[1])
    _wait_scatter(out_hbm, buf, wb0, ssems[1])
    _wait_scatter(out_hbm, buf, wb0, ssems[0])


@functools.partial(
    pl.kernel,
    out_type=jax.ShapeDtypeStruct((ROWS, HIDDEN), jnp.float32),
    mesh=plsc.VectorSubcoreMesh(core_axis_name="c", subcore_axis_name="s"),
    scratch_types=[
        pltpu.VMEM((IDS_PER_WORKER,), jnp.int32),
        pltpu.VMEM((2 * HIDDEN,), jnp.float32),
        pltpu.VMEM((4 * WB, HIDDEN), jnp.float32),
        pltpu.VMEM_SHARED((SEQ * HIDDEN,), jnp.float32),
        pltpu.SemaphoreType.DMA,
        pltpu.SemaphoreType.DMA,
        pltpu.SemaphoreType.DMA,
        pltpu.SemaphoreType.DMA,
    ],
)
def _emb_kernel(ids_hbm, tok_hbm, pos_hbm, out_hbm,
                idx_v, posbuf, buf, pos_sh, g0, g1, s0, s1):
    _emb_body(ids_hbm, tok_hbm, pos_hbm, out_hbm,
              idx_v, posbuf, buf, pos_sh, (g0, g1), (s0, s1))


def kernel(input_ids, token_table, position_table):
    ids_grouped = (
        input_ids.astype(jnp.int32)
        .reshape(NUM_WORKERS, WB, SEQ)
        .transpose(0, 2, 1)
        .reshape(-1)
    )
    out = _emb_kernel(ids_grouped, token_table, position_table.reshape(-1))
    return out.reshape(SEQ, BATCH, HIDDEN).transpose(1, 0, 2)
